# baseline (speedup 1.0000x reference)
import functools

import jax
import jax.numpy as jnp
from jax import lax
from jax.experimental import pallas as pl
from jax.experimental.pallas import tpu as pltpu
from jax.experimental.pallas import tpu_sc as plsc

_RHO = 0.5
_BETA = float(1.0 + (1.0 / 99.0) * (0.1 - 1.0))

_N, _K, _D = 4096, 32, 64
_H = 2
_NH = _N // _H
_B = _NH * _K
_NW = 32
_RPW = _B // _NW
_CH = 128
_NCH = _RPW // _CH
_NBUF = 4


def _gather_body(idx_hbm, table_hbm, out_hbm, idx_v, rows_v, in_sems, out_sems):
    cid = lax.axis_index("c")
    sid = lax.axis_index("s")
    wid = sid * 2 + cid
    row0 = wid * _RPW

    pltpu.sync_copy(idx_hbm.at[wid], idx_v)

    def g_start(chunk, slot):
        pltpu.make_async_copy(
            table_hbm.at[idx_v.at[chunk]], rows_v.at[slot], in_sems.at[slot]
        ).start()

    def g_wait(slot):
        pltpu.make_async_copy(
            table_hbm.at[idx_v.at[0]], rows_v.at[slot], in_sems.at[slot]
        ).wait()

    def o_start(chunk, slot):
        pltpu.make_async_copy(
            rows_v.at[slot], out_hbm.at[pl.ds(row0 + chunk * _CH, _CH)],
            out_sems.at[slot]
        ).start()

    def o_wait(slot):
        pltpu.make_async_copy(
            rows_v.at[slot], out_hbm.at[pl.ds(row0, _CH)], out_sems.at[slot]
        ).wait()

    for c in range(_NBUF):
        g_start(c, c)
    for c in range(_NCH):
        b = c % _NBUF
        g_wait(b)
        o_start(c, b)
        nxt = c + _NBUF
        if nxt < _NCH:
            o_wait(b)
            g_start(nxt, b)
    for c in range(_NCH - _NBUF, _NCH):
        o_wait(c % _NBUF)


def _sc_gather(idx3, table):
    mesh = plsc.VectorSubcoreMesh(core_axis_name="c", subcore_axis_name="s")
    fn = pl.kernel(
        _gather_body,
        mesh=mesh,
        out_type=jax.ShapeDtypeStruct((_B, _D), jnp.float32),
        scratch_types=[
            pltpu.VMEM((_NCH, _CH), jnp.int32),
            pltpu.VMEM((_NBUF, _CH, _D), jnp.float32),
            pltpu.SemaphoreType.DMA((_NBUF,)),
            pltpu.SemaphoreType.DMA((_NBUF,)),
        ],
        compiler_params=pltpu.CompilerParams(use_tc_tiling_on_sc=False),
    )
    return fn(idx3, table)


_BN = 256
_GRID = _NH // _BN


def _dense_body(p_ref, y_ref, lam_ref, x_ref, w_ref, t_ref,
                pn_ref, yn_ref, lamn_ref, en_ref):
    p = p_ref[...]
    t = t_ref[...].reshape(_BN, _K, _D)
    y_new = y_ref[...] + jnp.tanh(
        jnp.dot(x_ref[...], w_ref[...], preferred_element_type=jnp.float32))
    y_from_p = jnp.sum(p[:, :, None] * t, axis=1)
    xi = lam_ref[...] + _RHO * (y_new - y_from_p)
    scores = jnp.sum(t * xi[:, None, :], axis=2)
    logits = jnp.log(p + 1e-9) - _BETA * scores
    m = jnp.max(logits, axis=1, keepdims=True)
    e = jnp.exp(logits - m)
    p_new = e / jnp.sum(e, axis=1, keepdims=True)
    resid2 = y_new - jnp.sum(p_new[:, :, None] * t, axis=1)
    lam_new = lam_ref[...] + _RHO * resid2
    pn_ref[...] = p_new
    yn_ref[...] = y_new
    lamn_ref[...] = lam_new

    @pl.when(pl.program_id(0) == 0)
    def _():
        en_ref[0, 0] = 0.0

    en_ref[0, 0] += (0.5 * _RHO * jnp.sum(resid2 * resid2)
                     + jnp.sum(lam_new * resid2))


def _dense(P, Y, Lam, X, W_field, T):
    return pl.pallas_call(
        _dense_body,
        grid=(_GRID,),
        in_specs=[
            pl.BlockSpec((_BN, _K), lambda i: (i, 0)),
            pl.BlockSpec((_BN, _D), lambda i: (i, 0)),
            pl.BlockSpec((_BN, _D), lambda i: (i, 0)),
            pl.BlockSpec((_BN, _D), lambda i: (i, 0)),
            pl.BlockSpec((_D, _D), lambda i: (0, 0)),
            pl.BlockSpec((_BN * _K, _D), lambda i: (i, 0)),
        ],
        out_specs=[
            pl.BlockSpec((_BN, _K), lambda i: (i, 0)),
            pl.BlockSpec((_BN, _D), lambda i: (i, 0)),
            pl.BlockSpec((_BN, _D), lambda i: (i, 0)),
            pl.BlockSpec((1, 1), lambda i: (0, 0),
                         memory_space=pltpu.SMEM),
        ],
        out_shape=[
            jax.ShapeDtypeStruct((_NH, _K), jnp.float32),
            jax.ShapeDtypeStruct((_NH, _D), jnp.float32),
            jax.ShapeDtypeStruct((_NH, _D), jnp.float32),
            jax.ShapeDtypeStruct((1, 1), jnp.float32),
        ],
    )(P, Y, Lam, X, W_field, T)


def kernel(P, Y, Lam, X, M, W_field, Kset):
    idx = Kset.astype(jnp.int32).reshape(_H, _NW, _NCH, _CH)
    outs = []
    for h in range(_H):
        s = slice(h * _NH, (h + 1) * _NH)
        T = _sc_gather(idx[h], M)
        outs.append(_dense(P[s], Y[s], Lam[s], X[s], W_field, T))
    p_new = jnp.concatenate([o[0] for o in outs], axis=0)
    y_new = jnp.concatenate([o[1] for o in outs], axis=0)
    lam_new = jnp.concatenate([o[2] for o in outs], axis=0)
    energy = sum(o[3][0, 0] for o in outs)
    return (p_new, y_new, lam_new, energy)
